# baseline (speedup 1.0000x reference)
import jax
import jax.numpy as jnp
from jax.experimental import pallas as pl
from jax.experimental.pallas import tpu as pltpu
from jax.experimental.pallas import tpu_sc as plsc

TOPK = 2
BLK = 128
GW = 128


def _gather_rows(data, idx, width=256):
    M = idx.shape[0]
    N, D = data.shape
    R = D // width
    data_v = data.reshape(N * R, width)
    idx_v = (idx[:, None] * R + jnp.arange(R, dtype=jnp.int32)[None, :]).reshape(1, M * R)
    mesh = plsc.VectorSubcoreMesh(core_axis_name="c", subcore_axis_name="s")

    @pl.kernel(out_type=jax.ShapeDtypeStruct((M * R, width), data.dtype), mesh=mesh)
    def gather_kernel(x_hbm, i_hbm, o_hbm):
        def body(i_vmem, o_vmem):
            pltpu.sync_copy(x_hbm.at[i_vmem.at[0]], o_vmem)

        pltpu.emit_pipeline(
            body,
            grid=(M * R // GW,),
            in_specs=[pl.BlockSpec((1, GW), lambda i: (0, i))],
            out_specs=[pl.BlockSpec((GW, width), lambda i: (i, 0))],
            core_axis_name=("c", "s"),
            dimension_semantics=(pltpu.PARALLEL,),
        )(i_hbm, o_hbm)

    return gather_kernel(data_v, idx_v).reshape(M, D)


def _ffn_body(be_ref, x_ref, wg_ref, wu_ref, wd_ref, w_ref, y_ref):
    x = x_ref[...]
    g = jax.lax.dot_general(x, wg_ref[0], (((1,), (1,)), ((), ())),
                            preferred_element_type=jnp.float32)
    u = jax.lax.dot_general(x, wu_ref[0], (((1,), (1,)), ((), ())),
                            preferred_element_type=jnp.float32)
    h = jax.nn.silu(g) * u
    y = jax.lax.dot_general(h, wd_ref[0], (((1,), (1,)), ((), ())),
                            preferred_element_type=jnp.float32)
    w = w_ref[0, 0, :]
    y_ref[...] = y * w[:, None]


def _add_body(a_ref, b_ref, o_ref):
    o_ref[...] = a_ref[...] + b_ref[...]


def kernel(hidden_states, router_w, w_gate, w_up, w_down):
    B, S, D = hidden_states.shape
    E = router_w.shape[0]
    F = w_gate.shape[1]
    flat = hidden_states.reshape(-1, D)

    router_logits = flat @ router_w.T
    probs = jax.nn.softmax(router_logits, axis=-1)
    topv, topi = jax.lax.top_k(probs, TOPK)
    routing_weights = topv / jnp.sum(topv, axis=-1, keepdims=True)

    A = S * TOPK
    NB = (A + E * (BLK - 1)) // BLK
    NPAD = NB * BLK
    ids = topi.reshape(-1).astype(jnp.int32)
    wts = routing_weights.reshape(-1)
    onehot = (ids[:, None] == jnp.arange(E, dtype=jnp.int32)[None, :]).astype(jnp.int32)
    ranks_all = jnp.cumsum(onehot, axis=0) - onehot
    rank = jnp.sum(ranks_all * onehot, axis=1)
    counts = jnp.sum(onehot, axis=0)
    padded = ((counts + BLK - 1) // BLK) * BLK
    offsets = jnp.concatenate(
        [jnp.zeros(1, jnp.int32), jnp.cumsum(padded)[:-1].astype(jnp.int32)])
    pos = offsets[ids] + rank
    block_expert = (jnp.searchsorted(
        offsets, jnp.arange(NB, dtype=jnp.int32) * BLK, side="right") - 1)
    block_expert = jnp.clip(block_expert, 0, E - 1).astype(jnp.int32)
    tok_of_a = jnp.arange(A, dtype=jnp.int32) // TOPK
    row_token = jnp.zeros(NPAD, jnp.int32).at[pos].set(tok_of_a)
    w_sorted = jnp.zeros(NPAD, jnp.float32).at[pos].set(wts)

    x_sorted = _gather_rows(flat, row_token)

    grid_spec = pltpu.PrefetchScalarGridSpec(
        num_scalar_prefetch=1,
        grid=(NB,),
        in_specs=[
            pl.BlockSpec((BLK, D), lambda b, be: (b, 0)),
            pl.BlockSpec((1, F, D), lambda b, be: (be[b], 0, 0)),
            pl.BlockSpec((1, F, D), lambda b, be: (be[b], 0, 0)),
            pl.BlockSpec((1, D, F), lambda b, be: (be[b], 0, 0)),
            pl.BlockSpec((1, 1, BLK), lambda b, be: (b, 0, 0)),
        ],
        out_specs=pl.BlockSpec((BLK, D), lambda b, be: (b, 0)),
    )
    y = pl.pallas_call(
        _ffn_body,
        grid_spec=grid_spec,
        out_shape=jax.ShapeDtypeStruct((NPAD, D), jnp.float32),
        compiler_params=pltpu.CompilerParams(
            dimension_semantics=("arbitrary",),
            vmem_limit_bytes=100 * 1024 * 1024,
        ),
    )(block_expert, x_sorted, w_gate, w_up, w_down,
      w_sorted.reshape(NB, 1, BLK))

    pos_km = jnp.concatenate([pos[0::TOPK], pos[1::TOPK]])
    yga = _gather_rows(y, pos_km)

    SB = 256
    final = pl.pallas_call(
        _add_body,
        grid=(S // SB,),
        in_specs=[
            pl.BlockSpec((SB, D), lambda i: (i, 0)),
            pl.BlockSpec((SB, D), lambda i: (i + S // SB, 0)),
        ],
        out_specs=pl.BlockSpec((SB, D), lambda i: (i, 0)),
        out_shape=jax.ShapeDtypeStruct((S, D), jnp.float32),
    )(yga, yga)
    return final.reshape(B, S, D)


# speedup vs baseline: 1.0017x; 1.0017x over previous
import jax
import jax.numpy as jnp
from jax.experimental import pallas as pl
from jax.experimental.pallas import tpu as pltpu
from jax.experimental.pallas import tpu_sc as plsc

TOPK = 2
BLK = 128
GW = 128


def _gather_rows(data, idx, width=256):
    M = idx.shape[0]
    N, D = data.shape
    R = D // width
    data_v = data.reshape(N * R, width)
    idx_v = (idx[:, None] * R + jnp.arange(R, dtype=jnp.int32)[None, :]).reshape(1, M * R)
    mesh = plsc.VectorSubcoreMesh(core_axis_name="c", subcore_axis_name="s")

    @pl.kernel(out_type=jax.ShapeDtypeStruct((M * R, width), data.dtype), mesh=mesh)
    def gather_kernel(x_hbm, i_hbm, o_hbm):
        def body(i_vmem, o_vmem):
            pltpu.sync_copy(x_hbm.at[i_vmem.at[0]], o_vmem)

        pltpu.emit_pipeline(
            body,
            grid=(M * R // GW,),
            in_specs=[pl.BlockSpec((1, GW), lambda i: (0, i))],
            out_specs=[pl.BlockSpec((GW, width), lambda i: (i, 0))],
            core_axis_name=("c", "s"),
            dimension_semantics=(pltpu.PARALLEL,),
        )(i_hbm, o_hbm)

    return gather_kernel(data_v, idx_v).reshape(M, D)


def _ffn_body(be_ref, x_ref, wg_ref, wu_ref, wd_ref, w_ref, y_ref):
    x = x_ref[...]
    g = jax.lax.dot_general(x, wg_ref[0], (((1,), (1,)), ((), ())),
                            preferred_element_type=jnp.float32)
    u = jax.lax.dot_general(x, wu_ref[0], (((1,), (1,)), ((), ())),
                            preferred_element_type=jnp.float32)
    h = jax.nn.silu(g) * u
    y = jax.lax.dot_general(h, wd_ref[0], (((1,), (1,)), ((), ())),
                            preferred_element_type=jnp.float32)
    w = w_ref[0, 0, :]
    y_ref[...] = y * w[:, None]


def _add_body(a_ref, b_ref, o_ref):
    o_ref[...] = a_ref[...] + b_ref[...]


def kernel(hidden_states, router_w, w_gate, w_up, w_down):
    B, S, D = hidden_states.shape
    E = router_w.shape[0]
    F = w_gate.shape[1]
    flat = hidden_states.reshape(-1, D)

    router_logits = flat @ router_w.T
    probs = jax.nn.softmax(router_logits, axis=-1)
    topv, topi = jax.lax.top_k(probs, TOPK)
    routing_weights = topv / jnp.sum(topv, axis=-1, keepdims=True)

    A = S * TOPK
    NB = (A + E * (BLK - 1)) // BLK
    NPAD = NB * BLK
    ids = topi.reshape(-1).astype(jnp.int32)
    wts = routing_weights.reshape(-1)
    onehot = (ids[:, None] == jnp.arange(E, dtype=jnp.int32)[None, :]).astype(jnp.int32)
    ranks_all = jnp.cumsum(onehot, axis=0) - onehot
    rank = jnp.sum(ranks_all * onehot, axis=1)
    counts = jnp.sum(onehot, axis=0)
    padded = ((counts + BLK - 1) // BLK) * BLK
    offsets = jnp.concatenate(
        [jnp.zeros(1, jnp.int32), jnp.cumsum(padded)[:-1].astype(jnp.int32)])
    pos = offsets[ids] + rank
    block_expert = (jnp.searchsorted(
        offsets, jnp.arange(NB, dtype=jnp.int32) * BLK, side="right") - 1)
    block_expert = jnp.clip(block_expert, 0, E - 1).astype(jnp.int32)
    tok_of_a = jnp.arange(A, dtype=jnp.int32) // TOPK
    row_token = jnp.zeros(NPAD, jnp.int32).at[pos].set(tok_of_a)
    w_sorted = jnp.zeros(NPAD, jnp.float32).at[pos].set(wts)

    x_sorted = _gather_rows(flat, row_token)

    grid_spec = pltpu.PrefetchScalarGridSpec(
        num_scalar_prefetch=1,
        grid=(NB,),
        in_specs=[
            pl.BlockSpec((BLK, D), lambda b, be: (b, 0)),
            pl.BlockSpec((1, F, D), lambda b, be: (be[b], 0, 0)),
            pl.BlockSpec((1, F, D), lambda b, be: (be[b], 0, 0)),
            pl.BlockSpec((1, D, F), lambda b, be: (be[b], 0, 0)),
            pl.BlockSpec((1, 1, BLK), lambda b, be: (b, 0, 0)),
        ],
        out_specs=pl.BlockSpec((BLK, D), lambda b, be: (b, 0)),
    )
    y = pl.pallas_call(
        _ffn_body,
        grid_spec=grid_spec,
        out_shape=jax.ShapeDtypeStruct((NPAD, D), jnp.float32),
        compiler_params=pltpu.CompilerParams(
            dimension_semantics=("parallel",),
            vmem_limit_bytes=100 * 1024 * 1024,
        ),
    )(block_expert, x_sorted, w_gate, w_up, w_down,
      w_sorted.reshape(NB, 1, BLK))

    pos_km = jnp.concatenate([pos[0::TOPK], pos[1::TOPK]])
    yga = _gather_rows(y, pos_km)

    SB = 256
    final = pl.pallas_call(
        _add_body,
        grid=(S // SB,),
        in_specs=[
            pl.BlockSpec((SB, D), lambda i: (i, 0)),
            pl.BlockSpec((SB, D), lambda i: (i + S // SB, 0)),
        ],
        out_specs=pl.BlockSpec((SB, D), lambda i: (i, 0)),
        out_shape=jax.ShapeDtypeStruct((S, D), jnp.float32),
    )(yga, yga)
    return final.reshape(B, S, D)
